# Initial kernel scaffold; baseline (speedup 1.0000x reference)
import jax
import jax.numpy as jnp
from jax.experimental import pallas as pl


def kernel(x, edge_index, edge_weights, W1, b1, W2, b2, W3, b3, gamma1, beta1, gamma2, beta2):
    raise NotImplementedError("write your pallas kernel here")


# speedup vs baseline: 7.7725x; 7.7725x over previous
import functools

import jax
import jax.numpy as jnp
from jax import lax
from jax.experimental import pallas as pl
from jax.experimental.pallas import tpu as pltpu
from jax.experimental.pallas import tpu_sc as plsc

N = 10000
D = 128
DH = D // 2
E = 320000

NC = 2
NS = 16
NW = NC * NS
K = 128
CPW = 160
E_PAD = NS * CPW * K
N_PAD = 10240
RPT = N_PAD // NS

_mesh = plsc.VectorSubcoreMesh(core_axis_name="c", subcore_axis_name="s")


@functools.partial(
    pl.kernel,
    out_type=jax.ShapeDtypeStruct((NC, N_PAD), jnp.float32),
    mesh=_mesh,
    scratch_types=[
        pltpu.VMEM((CPW // 2, K), jnp.int32),
        pltpu.VMEM((K,), jnp.float32),
        pltpu.VMEM((RPT,), jnp.float32),
        pltpu.VMEM_SHARED((N_PAD,), jnp.float32),
    ],
)
def _sc_degree(dst_hbm, ones_hbm, zvec_hbm, deg_out, didx, ones_v, zv, deg_sh):
    c = lax.axis_index("c")
    s = lax.axis_index("s")
    wid = c * NS + s
    pltpu.sync_copy(zvec_hbm, zv)
    pltpu.sync_copy(zv, deg_sh.at[pl.ds(s * RPT, RPT)])
    pltpu.sync_copy(ones_hbm, ones_v)
    pltpu.sync_copy(dst_hbm.at[wid], didx)
    plsc.subcore_barrier()

    def body(ci, carry):
        pltpu.sync_copy(ones_v, deg_sh.at[didx.at[ci]], add=True)
        return carry

    lax.fori_loop(0, CPW // 2, body, 0)
    plsc.subcore_barrier()
    pltpu.sync_copy(deg_sh.at[pl.ds(s * RPT, RPT)],
                    deg_out.at[c, pl.ds(s * RPT, RPT)])


@functools.partial(
    pl.kernel,
    out_type=jax.ShapeDtypeStruct((NC, N_PAD, DH), jnp.float32),
    mesh=_mesh,
    scratch_types=[
        pltpu.VMEM((CPW, K), jnp.int32),
        pltpu.VMEM((CPW, K), jnp.int32),
        pltpu.VMEM((K, DH), jnp.float32),
        pltpu.VMEM((K, DH), jnp.float32),
        pltpu.VMEM_SHARED((N_PAD, DH), jnp.float32),
        pltpu.SemaphoreType.DMA,
    ],
    compiler_params=pltpu.CompilerParams(use_tc_tiling_on_sc=False),
)
def _sc_scatter(srcA_hbm, srcB_hbm, dst_hbm, y_hbm, zblk_hbm, z_out,
                sidx, didx, rows, zblk, z_sh, sem):
    c = lax.axis_index("c")
    s = lax.axis_index("s")
    pltpu.sync_copy(zblk_hbm, zblk)
    for j in range(RPT // K):
        pltpu.sync_copy(zblk, z_sh.at[pl.ds(s * RPT + j * K, K)])

    @pl.when(c == 0)
    def _():
        pltpu.sync_copy(srcA_hbm.at[s], sidx)

    @pl.when(c == 1)
    def _():
        pltpu.sync_copy(srcB_hbm.at[s], sidx)

    pltpu.sync_copy(dst_hbm.at[s], didx)
    plsc.subcore_barrier()

    def body(ci, carry):
        pltpu.async_copy(y_hbm.at[sidx.at[ci]], rows, sem).wait()
        pltpu.sync_copy(rows, z_sh.at[didx.at[ci]], add=True)
        return carry

    lax.fori_loop(0, CPW, body, 0)
    plsc.subcore_barrier()
    pltpu.sync_copy(z_sh.at[pl.ds(s * RPT, RPT)],
                    z_out.at[c, pl.ds(s * RPT, RPT)])


def _dinv_body(degp_ref, out_ref):
    d = degp_ref[0] + degp_ref[1] + 1.0
    out_ref[...] = 1.0 / jnp.sqrt(d)


def _tc_dinv(degp):
    return pl.pallas_call(
        _dinv_body,
        grid=(N_PAD // 128,),
        in_specs=[pl.BlockSpec((2, 128, 1), lambda i: (0, i, 0))],
        out_specs=pl.BlockSpec((128, 1), lambda i: (i, 0)),
        out_shape=jax.ShapeDtypeStruct((N_PAD, 1), jnp.float32),
    )(degp.reshape(2, N_PAD, 1))


def _mm_body(t_ref, w_ref, gp_ref, cc_ref, dinv_ref, out_ref):
    i = pl.program_id(0)
    h = t_ref[...] * gp_ref[...] + cc_ref[...]
    y = jnp.dot(h, w_ref[...], preferred_element_type=jnp.float32)
    y = y * dinv_ref[...]
    row = i * 128 + lax.broadcasted_iota(jnp.int32, (128, 1), 0)
    y = jnp.where(row < N, y, 0.0)
    out_ref[0] = y[:, :DH]
    out_ref[1] = y[:, DH:]


def _tc_matmul(t, w, gp, cc, dinv):
    return pl.pallas_call(
        _mm_body,
        grid=(N_PAD // 128,),
        in_specs=[
            pl.BlockSpec((128, D), lambda i: (i, 0)),
            pl.BlockSpec((D, D), lambda i: (0, 0)),
            pl.BlockSpec((1, D), lambda i: (0, 0)),
            pl.BlockSpec((1, D), lambda i: (0, 0)),
            pl.BlockSpec((128, 1), lambda i: (i, 0)),
        ],
        out_specs=pl.BlockSpec((2, 128, DH), lambda i: (0, i, 0)),
        out_shape=jax.ShapeDtypeStruct((NC, N_PAD, DH), jnp.float32),
    )(t, w, gp, cc, dinv)


def _post_body(z_ref, y_ref, b_ref, dinv_ref, t_ref, sums_ref, acc_s, acc_q):
    i = pl.program_id(0)

    @pl.when(i == 0)
    def _():
        acc_s[...] = jnp.zeros_like(acc_s)
        acc_q[...] = jnp.zeros_like(acc_q)

    row = i * 128 + lax.broadcasted_iota(jnp.int32, (128, 1), 0)
    for h in range(2):
        zz = z_ref[h] + y_ref[h]
        pre = dinv_ref[...] * zz + b_ref[:, h * DH:(h + 1) * DH]
        t = jnp.where(pre >= 0, pre, 0.01 * pre)
        t_ref[:, h * DH:(h + 1) * DH] = t
        tm = jnp.where(row < N, t, 0.0)
        acc_s[:, h * DH:(h + 1) * DH] += jnp.sum(tm, axis=0, keepdims=True)
        acc_q[:, h * DH:(h + 1) * DH] += jnp.sum(tm * tm, axis=0,
                                                 keepdims=True)

    @pl.when(i == N_PAD // 128 - 1)
    def _():
        sums_ref[...] = jnp.concatenate([acc_s[...], acc_q[...]], axis=0)


def _tc_post(z, y, b, dinv):
    return pl.pallas_call(
        _post_body,
        grid=(N_PAD // 128,),
        in_specs=[
            pl.BlockSpec((2, 128, DH), lambda i: (0, i, 0)),
            pl.BlockSpec((2, 128, DH), lambda i: (0, i, 0)),
            pl.BlockSpec((1, D), lambda i: (0, 0)),
            pl.BlockSpec((128, 1), lambda i: (i, 0)),
        ],
        out_specs=[
            pl.BlockSpec((128, D), lambda i: (i, 0)),
            pl.BlockSpec((2, D), lambda i: (0, 0)),
        ],
        out_shape=[
            jax.ShapeDtypeStruct((N_PAD, D), jnp.float32),
            jax.ShapeDtypeStruct((2, D), jnp.float32),
        ],
        scratch_shapes=[
            pltpu.VMEM((1, D), jnp.float32),
            pltpu.VMEM((1, D), jnp.float32),
        ],
    )(z, y, b, dinv)


def _final_body(z_ref, y_ref, b_ref, dinv_ref, out_ref):
    for h in range(2):
        zz = z_ref[h] + y_ref[h]
        pre = dinv_ref[...] * zz + b_ref[:, h * DH:(h + 1) * DH]
        out_ref[:, h * DH:(h + 1) * DH] = jnp.where(pre >= 0, pre, 0.01 * pre)


def _tc_final(z, y, b, dinv):
    return pl.pallas_call(
        _final_body,
        grid=(N_PAD // 128,),
        in_specs=[
            pl.BlockSpec((2, 128, DH), lambda i: (0, i, 0)),
            pl.BlockSpec((2, 128, DH), lambda i: (0, i, 0)),
            pl.BlockSpec((1, D), lambda i: (0, 0)),
            pl.BlockSpec((128, 1), lambda i: (i, 0)),
        ],
        out_specs=pl.BlockSpec((128, D), lambda i: (i, 0)),
        out_shape=jax.ShapeDtypeStruct((N_PAD, D), jnp.float32),
    )(z, y, b, dinv)


def kernel(x, edge_index, edge_weights, W1, b1, W2, b2, W3, b3,
           gamma1, beta1, gamma2, beta2):
    del edge_weights
    f32 = jnp.float32
    src = edge_index[0]
    dst = edge_index[1]
    pad = jnp.full((E_PAD - E,), N, jnp.int32)
    srcA = jnp.concatenate([src, pad]).reshape(NS, CPW, K)
    srcB = srcA + N_PAD
    dst3 = jnp.concatenate([dst, pad]).reshape(NS, CPW, K)
    dstw = dst3.reshape(NW, CPW // 2, K)

    ones_k = jnp.ones((K,), f32)
    zvec = jnp.zeros((RPT,), f32)
    zblk = jnp.zeros((K, DH), f32)
    x_pad = jnp.pad(x, ((0, N_PAD - N), (0, 0)))

    degp = _sc_degree(dstw, ones_k, zvec)
    dinv = _tc_dinv(degp)

    ones_row = jnp.ones((1, D), f32)
    zero_row = jnp.zeros((1, D), f32)

    def bn_coeffs(sums, gamma, beta):
        mean = sums[0] / N
        var = sums[1] / N - mean * mean
        gp = gamma / jnp.sqrt(var + 1e-5)
        cc = beta - mean * gp
        return gp.reshape(1, D), cc.reshape(1, D)

    def sc_layer(y):
        yflat = y.reshape(NC * N_PAD, DH)
        return _sc_scatter(srcA, srcB, dst3, yflat, zblk)

    y1 = _tc_matmul(x_pad, W1, ones_row, zero_row, dinv)
    z1 = sc_layer(y1)
    t1, sums1 = _tc_post(z1, y1, b1.reshape(1, D), dinv)
    gp1, cc1 = bn_coeffs(sums1, gamma1, beta1)

    y2 = _tc_matmul(t1, W2, gp1, cc1, dinv)
    z2 = sc_layer(y2)
    t2, sums2 = _tc_post(z2, y2, b2.reshape(1, D), dinv)
    gp2, cc2 = bn_coeffs(sums2, gamma2, beta2)

    y3 = _tc_matmul(t2, W3, gp2, cc2, dinv)
    z3 = sc_layer(y3)
    out = _tc_final(z3, y3, b3.reshape(1, D), dinv)
    return out[:N]


# speedup vs baseline: 9.6421x; 1.2405x over previous
import functools

import jax
import jax.numpy as jnp
from jax import lax
from jax.experimental import pallas as pl
from jax.experimental.pallas import tpu as pltpu
from jax.experimental.pallas import tpu_sc as plsc

N = 10000
D = 128
DH = D // 2
E = 320000

NC = 2
NS = 16
NW = NC * NS
K = 128
CPW = 160
E_PAD = NS * CPW * K
N_PAD = 10240
RPT = N_PAD // NS

_mesh = plsc.VectorSubcoreMesh(core_axis_name="c", subcore_axis_name="s")


@functools.partial(
    pl.kernel,
    out_type=jax.ShapeDtypeStruct((NC, N_PAD), jnp.float32),
    mesh=_mesh,
    scratch_types=[
        pltpu.VMEM((CPW // 2, K), jnp.int32),
        pltpu.VMEM((K,), jnp.float32),
        pltpu.VMEM((RPT,), jnp.float32),
        pltpu.VMEM_SHARED((N_PAD,), jnp.float32),
    ],
)
def _sc_degree(dst_hbm, ones_hbm, zvec_hbm, deg_out, didx, ones_v, zv, deg_sh):
    c = lax.axis_index("c")
    s = lax.axis_index("s")
    wid = c * NS + s
    pltpu.sync_copy(zvec_hbm, zv)
    pltpu.sync_copy(zv, deg_sh.at[pl.ds(s * RPT, RPT)])
    pltpu.sync_copy(ones_hbm, ones_v)
    pltpu.sync_copy(dst_hbm.at[wid], didx)
    plsc.subcore_barrier()

    def body(ci, carry):
        pltpu.sync_copy(ones_v, deg_sh.at[didx.at[ci]], add=True)
        return carry

    lax.fori_loop(0, CPW // 2, body, 0)
    plsc.subcore_barrier()
    pltpu.sync_copy(deg_sh.at[pl.ds(s * RPT, RPT)],
                    deg_out.at[c, pl.ds(s * RPT, RPT)])


@functools.partial(
    pl.kernel,
    out_type=jax.ShapeDtypeStruct((NC, N_PAD, DH), jnp.float32),
    mesh=_mesh,
    scratch_types=[
        pltpu.VMEM((CPW, K), jnp.int32),
        pltpu.VMEM((CPW, K), jnp.int32),
        pltpu.VMEM((K, DH), jnp.float32),
        pltpu.VMEM((K, DH), jnp.float32),
        pltpu.VMEM((K, DH), jnp.float32),
        pltpu.VMEM_SHARED((N_PAD, DH), jnp.float32),
        pltpu.SemaphoreType.DMA,
        pltpu.SemaphoreType.DMA,
    ],
    compiler_params=pltpu.CompilerParams(use_tc_tiling_on_sc=False),
)
def _sc_scatter(srcA_hbm, srcB_hbm, dst_hbm, y_hbm, zblk_hbm, z_out,
                sidx, didx, rowsA, rowsB, zblk, z_sh, semA, semB):
    c = lax.axis_index("c")
    s = lax.axis_index("s")
    pltpu.sync_copy(zblk_hbm, zblk)
    for j in range(RPT // K):
        pltpu.sync_copy(zblk, z_sh.at[pl.ds(s * RPT + j * K, K)])

    @pl.when(c == 0)
    def _():
        pltpu.sync_copy(srcA_hbm.at[s], sidx)

    @pl.when(c == 1)
    def _():
        pltpu.sync_copy(srcB_hbm.at[s], sidx)

    pltpu.sync_copy(dst_hbm.at[s], didx)
    plsc.subcore_barrier()

    def start(ci, buf, sem):
        pltpu.async_copy(y_hbm.at[sidx.at[ci]], buf, sem)

    def wait(buf, sem):
        pltpu.make_async_copy(y_hbm.at[sidx.at[0]], buf, sem).wait()

    start(0, rowsA, semA)

    def body(j, carry):
        c0 = 2 * j
        start(c0 + 1, rowsB, semB)
        wait(rowsA, semA)
        pltpu.sync_copy(rowsA, z_sh.at[didx.at[c0]], add=True)

        @pl.when(j < CPW // 2 - 1)
        def _():
            start(c0 + 2, rowsA, semA)

        wait(rowsB, semB)
        pltpu.sync_copy(rowsB, z_sh.at[didx.at[c0 + 1]], add=True)
        return carry

    lax.fori_loop(0, CPW // 2, body, 0)
    plsc.subcore_barrier()
    pltpu.sync_copy(z_sh.at[pl.ds(s * RPT, RPT)],
                    z_out.at[c, pl.ds(s * RPT, RPT)])


def _dinv_body(degp_ref, out_ref):
    d = degp_ref[0] + degp_ref[1] + 1.0
    out_ref[...] = 1.0 / jnp.sqrt(d)


def _tc_dinv(degp):
    return pl.pallas_call(
        _dinv_body,
        grid=(N_PAD // 128,),
        in_specs=[pl.BlockSpec((2, 128, 1), lambda i: (0, i, 0))],
        out_specs=pl.BlockSpec((128, 1), lambda i: (i, 0)),
        out_shape=jax.ShapeDtypeStruct((N_PAD, 1), jnp.float32),
    )(degp.reshape(2, N_PAD, 1))


def _mm_body(t_ref, w_ref, gp_ref, cc_ref, dinv_ref, out_ref):
    i = pl.program_id(0)
    h = t_ref[...] * gp_ref[...] + cc_ref[...]
    y = jnp.dot(h, w_ref[...], preferred_element_type=jnp.float32)
    y = y * dinv_ref[...]
    row = i * 128 + lax.broadcasted_iota(jnp.int32, (128, 1), 0)
    y = jnp.where(row < N, y, 0.0)
    out_ref[0] = y[:, :DH]
    out_ref[1] = y[:, DH:]


def _tc_matmul(t, w, gp, cc, dinv):
    return pl.pallas_call(
        _mm_body,
        grid=(N_PAD // 128,),
        in_specs=[
            pl.BlockSpec((128, D), lambda i: (i, 0)),
            pl.BlockSpec((D, D), lambda i: (0, 0)),
            pl.BlockSpec((1, D), lambda i: (0, 0)),
            pl.BlockSpec((1, D), lambda i: (0, 0)),
            pl.BlockSpec((128, 1), lambda i: (i, 0)),
        ],
        out_specs=pl.BlockSpec((2, 128, DH), lambda i: (0, i, 0)),
        out_shape=jax.ShapeDtypeStruct((NC, N_PAD, DH), jnp.float32),
    )(t, w, gp, cc, dinv)


def _post_body(z_ref, y_ref, b_ref, dinv_ref, t_ref, sums_ref, acc_s, acc_q):
    i = pl.program_id(0)

    @pl.when(i == 0)
    def _():
        acc_s[...] = jnp.zeros_like(acc_s)
        acc_q[...] = jnp.zeros_like(acc_q)

    row = i * 128 + lax.broadcasted_iota(jnp.int32, (128, 1), 0)
    for h in range(2):
        zz = z_ref[h] + y_ref[h]
        pre = dinv_ref[...] * zz + b_ref[:, h * DH:(h + 1) * DH]
        t = jnp.where(pre >= 0, pre, 0.01 * pre)
        t_ref[:, h * DH:(h + 1) * DH] = t
        tm = jnp.where(row < N, t, 0.0)
        acc_s[:, h * DH:(h + 1) * DH] += jnp.sum(tm, axis=0, keepdims=True)
        acc_q[:, h * DH:(h + 1) * DH] += jnp.sum(tm * tm, axis=0,
                                                 keepdims=True)

    @pl.when(i == N_PAD // 128 - 1)
    def _():
        sums_ref[...] = jnp.concatenate([acc_s[...], acc_q[...]], axis=0)


def _tc_post(z, y, b, dinv):
    return pl.pallas_call(
        _post_body,
        grid=(N_PAD // 128,),
        in_specs=[
            pl.BlockSpec((2, 128, DH), lambda i: (0, i, 0)),
            pl.BlockSpec((2, 128, DH), lambda i: (0, i, 0)),
            pl.BlockSpec((1, D), lambda i: (0, 0)),
            pl.BlockSpec((128, 1), lambda i: (i, 0)),
        ],
        out_specs=[
            pl.BlockSpec((128, D), lambda i: (i, 0)),
            pl.BlockSpec((2, D), lambda i: (0, 0)),
        ],
        out_shape=[
            jax.ShapeDtypeStruct((N_PAD, D), jnp.float32),
            jax.ShapeDtypeStruct((2, D), jnp.float32),
        ],
        scratch_shapes=[
            pltpu.VMEM((1, D), jnp.float32),
            pltpu.VMEM((1, D), jnp.float32),
        ],
    )(z, y, b, dinv)


def _final_body(z_ref, y_ref, b_ref, dinv_ref, out_ref):
    for h in range(2):
        zz = z_ref[h] + y_ref[h]
        pre = dinv_ref[...] * zz + b_ref[:, h * DH:(h + 1) * DH]
        out_ref[:, h * DH:(h + 1) * DH] = jnp.where(pre >= 0, pre, 0.01 * pre)


def _tc_final(z, y, b, dinv):
    return pl.pallas_call(
        _final_body,
        grid=(N_PAD // 128,),
        in_specs=[
            pl.BlockSpec((2, 128, DH), lambda i: (0, i, 0)),
            pl.BlockSpec((2, 128, DH), lambda i: (0, i, 0)),
            pl.BlockSpec((1, D), lambda i: (0, 0)),
            pl.BlockSpec((128, 1), lambda i: (i, 0)),
        ],
        out_specs=pl.BlockSpec((128, D), lambda i: (i, 0)),
        out_shape=jax.ShapeDtypeStruct((N_PAD, D), jnp.float32),
    )(z, y, b, dinv)


def kernel(x, edge_index, edge_weights, W1, b1, W2, b2, W3, b3,
           gamma1, beta1, gamma2, beta2):
    del edge_weights
    f32 = jnp.float32
    src = edge_index[0]
    dst = edge_index[1]
    pad = jnp.full((E_PAD - E,), N, jnp.int32)
    srcA = jnp.concatenate([src, pad]).reshape(NS, CPW, K)
    srcB = srcA + N_PAD
    dst3 = jnp.concatenate([dst, pad]).reshape(NS, CPW, K)
    dstw = dst3.reshape(NW, CPW // 2, K)

    ones_k = jnp.ones((K,), f32)
    zvec = jnp.zeros((RPT,), f32)
    zblk = jnp.zeros((K, DH), f32)
    x_pad = jnp.pad(x, ((0, N_PAD - N), (0, 0)))

    degp = _sc_degree(dstw, ones_k, zvec)
    dinv = _tc_dinv(degp)

    ones_row = jnp.ones((1, D), f32)
    zero_row = jnp.zeros((1, D), f32)

    def bn_coeffs(sums, gamma, beta):
        mean = sums[0] / N
        var = sums[1] / N - mean * mean
        gp = gamma / jnp.sqrt(var + 1e-5)
        cc = beta - mean * gp
        return gp.reshape(1, D), cc.reshape(1, D)

    def sc_layer(y):
        yflat = y.reshape(NC * N_PAD, DH)
        return _sc_scatter(srcA, srcB, dst3, yflat, zblk)

    y1 = _tc_matmul(x_pad, W1, ones_row, zero_row, dinv)
    z1 = sc_layer(y1)
    t1, sums1 = _tc_post(z1, y1, b1.reshape(1, D), dinv)
    gp1, cc1 = bn_coeffs(sums1, gamma1, beta1)

    y2 = _tc_matmul(t1, W2, gp1, cc1, dinv)
    z2 = sc_layer(y2)
    t2, sums2 = _tc_post(z2, y2, b2.reshape(1, D), dinv)
    gp2, cc2 = bn_coeffs(sums2, gamma2, beta2)

    y3 = _tc_matmul(t2, W3, gp2, cc2, dinv)
    z3 = sc_layer(y3)
    out = _tc_final(z3, y3, b3.reshape(1, D), dinv)
    return out[:N]
